# baseline (speedup 1.0000x reference)
import functools

import jax
import jax.numpy as jnp
from jax import lax
from jax.experimental import pallas as pl
from jax.experimental.pallas import tpu as pltpu
from jax.experimental.pallas import tpu_sc as plsc

VOCAB_ROWS = 100000
DIM = 64

_NUM_CORES = 2
_NUM_SUBCORES = 16
_NUM_WORKERS = _NUM_CORES * _NUM_SUBCORES
_WCHUNK = 3128
_LAST = VOCAB_ROWS - (_NUM_WORKERS - 1) * _WCHUNK
_NBUF = 4
_CROWS = 248
_AHEAD = 1


def _chunk_sizes(total):
    sizes = []
    while total > 0:
        sizes.append(min(_CROWS, total))
        total -= sizes[-1]
    return sizes


@functools.partial(
    pl.kernel,
    mesh=plsc.VectorSubcoreMesh(core_axis_name="c", subcore_axis_name="s"),
    out_type=jax.ShapeDtypeStruct((VOCAB_ROWS, DIM), jnp.float32),
    compiler_params=pltpu.CompilerParams(use_tc_tiling_on_sc=True),
    scratch_types=(
        [pltpu.VMEM((_CROWS, DIM), jnp.float32)] * _NBUF
        + [pltpu.SemaphoreType.DMA] * (2 * _NBUF)
    ),
)
def _copy_kernel(in_hbm, out_hbm, *scratch):
    bufs = scratch[:_NBUF]
    sin = scratch[_NBUF:2 * _NBUF]
    sout = scratch[2 * _NBUF:]
    wid = lax.axis_index("s") * _NUM_CORES + lax.axis_index("c")
    base = pl.multiple_of(wid * _WCHUNK, 8)

    def copy_range(start, total):
        sizes = _chunk_sizes(total)
        n = len(sizes)
        offs = []
        off = 0
        for sz in sizes:
            offs.append(off)
            off += sz
        h_in = [None] * _NBUF
        h_out = [None] * _NBUF
        for j in range(min(_AHEAD, n)):
            h_in[j] = pltpu.async_copy(
                in_hbm.at[pl.ds(start + offs[j], sizes[j])],
                bufs[j].at[pl.ds(0, sizes[j])], sin[j])
        for i in range(n):
            b = i % _NBUF
            h_in[b].wait()
            h_out[b] = pltpu.async_copy(
                bufs[b].at[pl.ds(0, sizes[i])],
                out_hbm.at[pl.ds(start + offs[i], sizes[i])], sout[b])
            j = i + _AHEAD
            if j < n:
                bj = j % _NBUF
                if h_out[bj] is not None:
                    h_out[bj].wait()
                h_in[bj] = pltpu.async_copy(
                    in_hbm.at[pl.ds(start + offs[j], sizes[j])],
                    bufs[bj].at[pl.ds(0, sizes[j])], sin[bj])
        for k in range(max(0, n - _NBUF), n):
            h_out[k % _NBUF].wait()

    @pl.when(wid < _NUM_WORKERS - 1)
    def _():
        copy_range(base, _WCHUNK)

    @pl.when(wid == _NUM_WORKERS - 1)
    def _():
        copy_range((_NUM_WORKERS - 1) * _WCHUNK, _LAST)


def kernel(embeddings):
    return _copy_kernel(embeddings)


# speedup vs baseline: 1.0344x; 1.0344x over previous
import functools

import jax
import jax.numpy as jnp
from jax import lax
from jax.experimental import pallas as pl
from jax.experimental.pallas import tpu as pltpu
from jax.experimental.pallas import tpu_sc as plsc

VOCAB_ROWS = 100000
DIM = 64

_NUM_CORES = 2
_NUM_SUBCORES = 16
_NUM_WORKERS = _NUM_CORES * _NUM_SUBCORES
_WCHUNK = 3128
_LAST = VOCAB_ROWS - (_NUM_WORKERS - 1) * _WCHUNK
_NBUF = 6
_CROWS = 168
_AHEAD = 2


def _chunk_sizes(total):
    sizes = []
    while total > 0:
        sizes.append(min(_CROWS, total))
        total -= sizes[-1]
    return sizes


@functools.partial(
    pl.kernel,
    mesh=plsc.VectorSubcoreMesh(core_axis_name="c", subcore_axis_name="s"),
    out_type=jax.ShapeDtypeStruct((VOCAB_ROWS, DIM), jnp.float32),
    compiler_params=pltpu.CompilerParams(use_tc_tiling_on_sc=True),
    scratch_types=(
        [pltpu.VMEM((_CROWS, DIM), jnp.float32)] * _NBUF
        + [pltpu.SemaphoreType.DMA] * (2 * _NBUF)
    ),
)
def _copy_kernel(in_hbm, out_hbm, *scratch):
    bufs = scratch[:_NBUF]
    sin = scratch[_NBUF:2 * _NBUF]
    sout = scratch[2 * _NBUF:]
    wid = lax.axis_index("s") * _NUM_CORES + lax.axis_index("c")
    base = pl.multiple_of(wid * _WCHUNK, 8)

    def copy_range(start, total):
        sizes = _chunk_sizes(total)
        n = len(sizes)
        offs = []
        off = 0
        for sz in sizes:
            offs.append(off)
            off += sz
        h_in = [None] * _NBUF
        h_out = [None] * _NBUF
        for j in range(min(_AHEAD, n)):
            h_in[j] = pltpu.async_copy(
                in_hbm.at[pl.ds(start + offs[j], sizes[j])],
                bufs[j].at[pl.ds(0, sizes[j])], sin[j])
        for i in range(n):
            b = i % _NBUF
            h_in[b].wait()
            h_out[b] = pltpu.async_copy(
                bufs[b].at[pl.ds(0, sizes[i])],
                out_hbm.at[pl.ds(start + offs[i], sizes[i])], sout[b])
            j = i + _AHEAD
            if j < n:
                bj = j % _NBUF
                if h_out[bj] is not None:
                    h_out[bj].wait()
                h_in[bj] = pltpu.async_copy(
                    in_hbm.at[pl.ds(start + offs[j], sizes[j])],
                    bufs[bj].at[pl.ds(0, sizes[j])], sin[bj])
        for k in range(max(0, n - _NBUF), n):
            h_out[k % _NBUF].wait()

    @pl.when(wid < _NUM_WORKERS - 1)
    def _():
        copy_range(base, _WCHUNK)

    @pl.when(wid == _NUM_WORKERS - 1)
    def _():
        copy_range((_NUM_WORKERS - 1) * _WCHUNK, _LAST)


def kernel(embeddings):
    return _copy_kernel(embeddings)
